# Initial kernel scaffold; baseline (speedup 1.0000x reference)
import jax
import jax.numpy as jnp
from jax.experimental import pallas as pl


def kernel(xx, edge_index, parent_index, W, W1, b1, W2, b2):
    raise NotImplementedError("write your pallas kernel here")


# speedup vs baseline: 31.9678x; 31.9678x over previous
import functools

import jax
import jax.numpy as jnp
from jax import lax
from jax.experimental import pallas as pl
from jax.experimental.pallas import tpu as pltpu
from jax.experimental.pallas import tpu_sc as plsc

N = 10000
E = 320000
C = 128
NC = 2
NS = 16
NW = NC * NS
EPW = E // NW
CH = 80
NCH = EPW // CH
NPAD = 10240
RPT = NPAD // NS

_mesh = plsc.VectorSubcoreMesh(core_axis_name="c", subcore_axis_name="s")


@functools.partial(
    pl.kernel,
    out_type=jax.ShapeDtypeStruct((NC, NPAD), jnp.float32),
    mesh=_mesh,
    scratch_types=[
        pltpu.VMEM((NCH, CH), jnp.int32),
        pltpu.VMEM((CH,), jnp.float32),
        pltpu.VMEM((RPT,), jnp.float32),
        pltpu.VMEM_SHARED((NPAD,), jnp.float32),
    ],
)
def _sc_hist(dst_hbm, deg_hbm, idx_v, ones_v, zb_v, deg_sh):
    c = lax.axis_index("c")
    s = lax.axis_index("s")
    wid = c * NS + s
    for i in range(CH // 16):
        ones_v[pl.ds(16 * i, 16)] = jnp.ones((16,), jnp.float32)

    def zinit(j, carry):
        zb_v[pl.ds(j * 16, 16)] = jnp.zeros((16,), jnp.float32)
        return carry

    lax.fori_loop(0, RPT // 16, zinit, 0)
    pltpu.sync_copy(zb_v, deg_sh.at[pl.ds(s * RPT, RPT)])
    plsc.subcore_barrier()
    pltpu.sync_copy(dst_hbm.at[wid], idx_v)

    def body(j, carry):
        pltpu.sync_copy(ones_v, deg_sh.at[idx_v.at[j]], add=True)
        return carry

    lax.fori_loop(0, NCH, body, 0)
    plsc.subcore_barrier()
    pltpu.sync_copy(deg_sh.at[pl.ds(s * RPT, RPT)],
                    deg_hbm.at[c, pl.ds(s * RPT, RPT)])


@functools.partial(
    pl.kernel,
    out_type=jax.ShapeDtypeStruct((NC, NPAD, C), jnp.float32),
    mesh=_mesh,
    scratch_types=[
        pltpu.VMEM((NCH, CH), jnp.int32),
        pltpu.VMEM((NCH, CH), jnp.int32),
        pltpu.VMEM((CH, C), jnp.float32),
        pltpu.VMEM_SHARED((NPAD, C), jnp.float32),
        pltpu.SemaphoreType.DMA,
    ],
)
def _sc_scatter(y_hbm, src_hbm, dst_hbm, ap_hbm,
                src_v, dst_v, stage_v, a_sh, gsem):
    c = lax.axis_index("c")
    s = lax.axis_index("s")
    wid = c * NS + s

    def zinit(j, carry):
        for k in range(C // 16):
            stage_v[j, pl.ds(16 * k, 16)] = jnp.zeros((16,), jnp.float32)
        return carry

    lax.fori_loop(0, CH, zinit, 0)
    for k in range(RPT // CH):
        pltpu.sync_copy(stage_v, a_sh.at[pl.ds(s * RPT + k * CH, CH)])
    plsc.subcore_barrier()

    pltpu.sync_copy(src_hbm.at[wid], src_v)
    pltpu.sync_copy(dst_hbm.at[wid], dst_v)

    def body(j, carry):
        pltpu.async_copy(y_hbm.at[src_v.at[j]], stage_v, gsem).wait()
        pltpu.sync_copy(stage_v, a_sh.at[dst_v.at[j]], add=True)
        return carry

    lax.fori_loop(0, NCH, body, 0)
    plsc.subcore_barrier()
    pltpu.sync_copy(a_sh.at[pl.ds(s * RPT, RPT)],
                    ap_hbm.at[c, pl.ds(s * RPT, RPT)])


def _scale_body(deg_ref, xx_ref, y_ref):
    deg = deg_ref[:, 0:1] + deg_ref[:, 1:2] + 1.0
    y_ref[...] = xx_ref[...] * lax.rsqrt(deg)


def _tc_scale(xx, degt):
    rb = 1000
    return pl.pallas_call(
        _scale_body,
        grid=(N // rb,),
        in_specs=[
            pl.BlockSpec((rb, NC), lambda i: (i, 0)),
            pl.BlockSpec((rb, C), lambda i: (i, 0)),
        ],
        out_specs=pl.BlockSpec((rb, C), lambda i: (i, 0)),
        out_shape=jax.ShapeDtypeStruct((N, C), jnp.float32),
    )(degt, xx)


def _dense_body(ap_ref, y_ref, xx_ref, deg_ref, w_ref, w1_ref, b1_ref,
                w2_ref, b2_ref, out_ref):
    deg = deg_ref[:, 0:1] + deg_ref[:, 1:2] + 1.0
    dinv = lax.rsqrt(deg)
    a = (ap_ref[0] + ap_ref[1] + y_ref[...]) * dinv
    ww1 = jnp.dot(w_ref[...], w1_ref[...], preferred_element_type=jnp.float32)
    h = jnp.maximum(
        jnp.dot(a, ww1, preferred_element_type=jnp.float32) + b1_ref[...], 0.0)
    up = jnp.dot(h, w2_ref[...], preferred_element_type=jnp.float32) + b2_ref[...]
    out_ref[...] = xx_ref[...] + up


def _tc_dense(ap, y, xx, degt, W, W1, b1, W2, b2):
    rb = 1000
    return pl.pallas_call(
        _dense_body,
        grid=(N // rb,),
        in_specs=[
            pl.BlockSpec((NC, rb, C), lambda i: (0, i, 0)),
            pl.BlockSpec((rb, C), lambda i: (i, 0)),
            pl.BlockSpec((rb, C), lambda i: (i, 0)),
            pl.BlockSpec((rb, NC), lambda i: (i, 0)),
            pl.BlockSpec((C, 3 * C), lambda i: (0, 0)),
            pl.BlockSpec((3 * C, 32), lambda i: (0, 0)),
            pl.BlockSpec((1, 32), lambda i: (0, 0)),
            pl.BlockSpec((32, C), lambda i: (0, 0)),
            pl.BlockSpec((1, C), lambda i: (0, 0)),
        ],
        out_specs=pl.BlockSpec((rb, C), lambda i: (i, 0)),
        out_shape=jax.ShapeDtypeStruct((N, C), jnp.float32),
    )(ap, y, xx, degt, W, W1, b1, W2, b2)


def kernel(xx, edge_index, parent_index, W, W1, b1, W2, b2):
    ei = edge_index.astype(jnp.int32)
    src_r = ei[0].reshape(NW, NCH, CH)
    dst_r = ei[1].reshape(NW, NCH, CH)
    degp = _sc_hist(dst_r)
    degt = degp.T
    y = _tc_scale(xx, degt)
    ap = _sc_scatter(y, src_r, dst_r)
    return _tc_dense(ap, y, xx, degt, W, W1,
                     b1.reshape(1, -1), W2, b2.reshape(1, -1))
